# baseline (speedup 1.0000x reference)
import functools
import math

import jax
import jax.numpy as jnp
from jax.experimental import pallas as pl
from jax.experimental.pallas import tpu as pltpu

ALPHA = 0.2


def _stage1_kernel(n, rblk, x_ref, w_ref, a1_ref, a2_ref,
                   h_ref, u_ref, v_ref, va_ref):
    r = pl.program_id(0)
    rows = r * rblk + jax.lax.broadcasted_iota(jnp.int32, (rblk, 1), 0)
    valid = rows < n
    h = jnp.dot(x_ref[...], w_ref[...], preferred_element_type=jnp.float32)
    h = jnp.where(valid, h, 0.0)
    h_ref[:, : h.shape[1]] = h
    if h_ref.shape[1] > h.shape[1]:
        h_ref[:, h.shape[1] :] = jnp.broadcast_to(
            jnp.where(valid, 1.0, 0.0), (rblk, h_ref.shape[1] - h.shape[1]))
    s1 = jax.lax.dot_general(h, a1_ref[...], (((1,), (1,)), ((), ())),
                             preferred_element_type=jnp.float32)
    u = jnp.where(valid, jnp.exp(-s1), 0.0)
    ua = jnp.where(valid, jnp.exp(-ALPHA * s1), 0.0)
    lane = jax.lax.broadcasted_iota(jnp.int32, (1, u_ref.shape[1]), 1)
    u_ref[...] = jnp.where(lane == 0, u, jnp.where(lane == 1, ua, 0.0))
    s2 = jax.lax.dot_general(a2_ref[...], h, (((1,), (1,)), ((), ())),
                             preferred_element_type=jnp.float32)
    cols = r * rblk + jax.lax.broadcasted_iota(jnp.int32, (1, rblk), 1)
    cvalid = cols < n
    v = jnp.where(cvalid, jnp.exp(-s2), 0.0)
    va = jnp.where(cvalid, jnp.exp(-ALPHA * s2), 0.0)
    v_ref[...] = jnp.broadcast_to(v, v_ref.shape)
    va_ref[...] = jnp.broadcast_to(va, va_ref.shape)


def _stage2_kernel(nc, cblk, out_f, adj_ref, h_ref, u_ref,
                   v_ref, va_ref, out_ref):
    c = pl.program_id(1)
    u = u_ref[:, 0:1]
    ua = u_ref[:, 1:2]
    v = v_ref[0:1, :]
    va = va_ref[0:1, :]
    e = jnp.where(adj_ref[...] > 0.0,
                  jnp.minimum(u * v, ua * va), 0.0)
    h_blk = h_ref[pl.ds(c * cblk, cblk), :]
    partial = jnp.dot(e, h_blk, preferred_element_type=jnp.float32)

    if nc == 1:
        hp = partial[:, :out_f] / partial[:, out_f : out_f + 1]
        out_ref[...] = jnp.where(hp > 0, hp, jnp.exp(hp) - 1.0)
        return

    @pl.when(c == 0)
    def _init():
        out_ref[...] = partial

    @pl.when(c > 0)
    def _acc():
        out_ref[...] = out_ref[...] + partial

    @pl.when(c == nc - 1)
    def _fin():
        acc = out_ref[...]
        hp = acc[:, :out_f] / acc[:, out_f : out_f + 1]
        out_ref[:, :out_f] = jnp.where(hp > 0, hp, jnp.exp(hp) - 1.0)


def _stage2_split_kernel(half, out_f, adj0_ref, adj1_ref, h_ref, u_ref,
                         v_ref, va_ref, out_ref):
    u = u_ref[:, 0:1]
    ua = u_ref[:, 1:2]
    acc = None
    for k, adj_ref in ((0, adj0_ref), (1, adj1_ref)):
        v = v_ref[0:1, pl.ds(k * half, half)]
        va = va_ref[0:1, pl.ds(k * half, half)]
        e = jnp.where(adj_ref[...] > 0.0,
                      jnp.minimum(u * v, ua * va), 0.0)
        h_blk = h_ref[pl.ds(k * half, half), :]
        d = jnp.dot(e, h_blk, preferred_element_type=jnp.float32)
        acc = d if acc is None else acc + d
    hp = acc[:, :out_f] / acc[:, out_f : out_f + 1]
    out_ref[...] = jnp.where(hp > 0, hp, jnp.exp(hp) - 1.0)


def _gat_forward(x, adj, w, a, rblk1=512, rblk=320, cblk=10240,
                 csplit=False):
    n, in_f = x.shape
    out_f = w.shape[1]
    hw = out_f + 128
    step = math.lcm(rblk, cblk, rblk1)
    npad = ((n + step - 1) // step) * step
    nr = -(-n // rblk)
    nc = npad // cblk
    a1 = a[:, :out_f]
    a2 = a[:, out_f:]

    h, u, v, va = pl.pallas_call(
        functools.partial(_stage1_kernel, n, rblk1),
        grid=(npad // rblk1,),
        in_specs=[
            pl.BlockSpec((rblk1, in_f), lambda r: (r, 0)),
            pl.BlockSpec((in_f, out_f), lambda r: (0, 0)),
            pl.BlockSpec((1, out_f), lambda r: (0, 0)),
            pl.BlockSpec((1, out_f), lambda r: (0, 0)),
        ],
        out_specs=[
            pl.BlockSpec((rblk1, hw), lambda r: (r, 0)),
            pl.BlockSpec((rblk1, 128), lambda r: (r, 0)),
            pl.BlockSpec((8, rblk1), lambda r: (0, r)),
            pl.BlockSpec((8, rblk1), lambda r: (0, r)),
        ],
        out_shape=[
            jax.ShapeDtypeStruct((npad, hw), jnp.float32),
            jax.ShapeDtypeStruct((npad, 128), jnp.float32),
            jax.ShapeDtypeStruct((8, npad), jnp.float32),
            jax.ShapeDtypeStruct((8, npad), jnp.float32),
        ],
    )(x, w, a1, a2)

    if csplit:
        assert nc == 1
        half = cblk // 2
        out = pl.pallas_call(
            functools.partial(_stage2_split_kernel, half, out_f),
            grid=(nr,),
            in_specs=[
                pl.BlockSpec((rblk, half), lambda r: (r, 0)),
                pl.BlockSpec((rblk, half), lambda r: (r, 1)),
                pl.BlockSpec((npad, hw), lambda r: (0, 0)),
                pl.BlockSpec((rblk, 128), lambda r: (r, 0)),
                pl.BlockSpec((8, cblk), lambda r: (0, 0)),
                pl.BlockSpec((8, cblk), lambda r: (0, 0)),
            ],
            out_specs=pl.BlockSpec((rblk, out_f), lambda r: (r, 0)),
            out_shape=jax.ShapeDtypeStruct((n, out_f), jnp.float32),
            compiler_params=pltpu.CompilerParams(
                dimension_semantics=("parallel",)),
        )(adj, adj, h, u, v, va)
        return out

    out_w = out_f if nc == 1 else hw
    out = pl.pallas_call(
        functools.partial(_stage2_kernel, nc, cblk, out_f),
        grid=(nr, nc),
        in_specs=[
            pl.BlockSpec((rblk, cblk), lambda r, c: (r, c)),
            pl.BlockSpec((npad, hw), lambda r, c: (0, 0)),
            pl.BlockSpec((rblk, 128), lambda r, c: (r, 0)),
            pl.BlockSpec((8, cblk), lambda r, c: (0, c)),
            pl.BlockSpec((8, cblk), lambda r, c: (0, c)),
        ],
        out_specs=pl.BlockSpec((rblk, out_w), lambda r, c: (r, 0)),
        out_shape=jax.ShapeDtypeStruct(
            (n if nc == 1 else npad, out_w), jnp.float32),
        compiler_params=pltpu.CompilerParams(
            dimension_semantics=("parallel", "arbitrary")),
    )(adj, h, u, v, va)
    return out if nc == 1 else out[:n, :out_f]


def kernel(input, adj, W, a):
    return _gat_forward(input, adj, W, a, rblk=256)


# speedup vs baseline: 1.0329x; 1.0329x over previous
import functools
import math

import jax
import jax.numpy as jnp
from jax.experimental import pallas as pl
from jax.experimental.pallas import tpu as pltpu

ALPHA = 0.2


def _stage1_kernel(n, rblk, x_ref, w_ref, a1_ref, a2_ref,
                   h_ref, u_ref, v_ref, va_ref):
    r = pl.program_id(0)
    rows = r * rblk + jax.lax.broadcasted_iota(jnp.int32, (rblk, 1), 0)
    valid = rows < n
    h = jnp.dot(x_ref[...], w_ref[...], preferred_element_type=jnp.float32)
    h = jnp.where(valid, h, 0.0)
    h_ref[:, : h.shape[1]] = h
    if h_ref.shape[1] > h.shape[1]:
        h_ref[:, h.shape[1] :] = jnp.broadcast_to(
            jnp.where(valid, 1.0, 0.0), (rblk, h_ref.shape[1] - h.shape[1]))
    s1 = jax.lax.dot_general(h, a1_ref[...], (((1,), (1,)), ((), ())),
                             preferred_element_type=jnp.float32)
    u = jnp.where(valid, jnp.exp(-s1), 0.0)
    ua = jnp.where(valid, jnp.exp(-ALPHA * s1), 0.0)
    lane = jax.lax.broadcasted_iota(jnp.int32, (1, u_ref.shape[1]), 1)
    u_ref[...] = jnp.where(lane == 0, u, jnp.where(lane == 1, ua, 0.0))
    s2 = jax.lax.dot_general(a2_ref[...], h, (((1,), (1,)), ((), ())),
                             preferred_element_type=jnp.float32)
    cols = r * rblk + jax.lax.broadcasted_iota(jnp.int32, (1, rblk), 1)
    cvalid = cols < n
    v = jnp.where(cvalid, jnp.exp(-s2), 0.0)
    va = jnp.where(cvalid, jnp.exp(-ALPHA * s2), 0.0)
    v_ref[...] = jnp.broadcast_to(v, v_ref.shape)
    va_ref[...] = jnp.broadcast_to(va, va_ref.shape)


def _stage2_kernel(nc, cblk, out_f, adj_ref, h_ref, u_ref,
                   v_ref, va_ref, out_ref):
    c = pl.program_id(1)
    u = u_ref[:, 0:1]
    ua = u_ref[:, 1:2]
    v = v_ref[0:1, :]
    va = va_ref[0:1, :]
    e = jnp.where(adj_ref[...] > 0.0,
                  jnp.minimum(u * v, ua * va), 0.0)
    h_blk = h_ref[pl.ds(c * cblk, cblk), :]
    partial = jnp.dot(e, h_blk, preferred_element_type=jnp.float32)

    if nc == 1:
        hp = partial[:, :out_f] / partial[:, out_f : out_f + 1]
        out_ref[...] = jnp.where(hp > 0, hp, jnp.exp(hp) - 1.0)
        return

    @pl.when(c == 0)
    def _init():
        out_ref[...] = partial

    @pl.when(c > 0)
    def _acc():
        out_ref[...] = out_ref[...] + partial

    @pl.when(c == nc - 1)
    def _fin():
        acc = out_ref[...]
        hp = acc[:, :out_f] / acc[:, out_f : out_f + 1]
        out_ref[:, :out_f] = jnp.where(hp > 0, hp, jnp.exp(hp) - 1.0)


def _stage2_split_kernel(half, out_f, adj0_ref, adj1_ref, h_ref, u_ref,
                         v_ref, va_ref, out_ref):
    u = u_ref[:, 0:1]
    ua = u_ref[:, 1:2]
    acc = None
    for k, adj_ref in ((0, adj0_ref), (1, adj1_ref)):
        v = v_ref[0:1, pl.ds(k * half, half)]
        va = va_ref[0:1, pl.ds(k * half, half)]
        e = jnp.where(adj_ref[...] > 0.0,
                      jnp.minimum(u * v, ua * va), 0.0)
        h_blk = h_ref[pl.ds(k * half, half), :]
        d = jnp.dot(e, h_blk, preferred_element_type=jnp.float32)
        acc = d if acc is None else acc + d
    hp = acc[:, :out_f] / acc[:, out_f : out_f + 1]
    out_ref[...] = jnp.where(hp > 0, hp, jnp.exp(hp) - 1.0)


def _gat_forward(x, adj, w, a, rblk1=512, rblk=320, cblk=10240,
                 csplit=False):
    n, in_f = x.shape
    out_f = w.shape[1]
    hw = out_f + 8
    step = math.lcm(rblk, cblk, rblk1)
    npad = ((n + step - 1) // step) * step
    nr = -(-n // rblk)
    nc = npad // cblk
    a1 = a[:, :out_f]
    a2 = a[:, out_f:]

    h, u, v, va = pl.pallas_call(
        functools.partial(_stage1_kernel, n, rblk1),
        grid=(npad // rblk1,),
        in_specs=[
            pl.BlockSpec((rblk1, in_f), lambda r: (r, 0)),
            pl.BlockSpec((in_f, out_f), lambda r: (0, 0)),
            pl.BlockSpec((1, out_f), lambda r: (0, 0)),
            pl.BlockSpec((1, out_f), lambda r: (0, 0)),
        ],
        out_specs=[
            pl.BlockSpec((rblk1, hw), lambda r: (r, 0)),
            pl.BlockSpec((rblk1, 128), lambda r: (r, 0)),
            pl.BlockSpec((8, rblk1), lambda r: (0, r)),
            pl.BlockSpec((8, rblk1), lambda r: (0, r)),
        ],
        out_shape=[
            jax.ShapeDtypeStruct((npad, hw), jnp.float32),
            jax.ShapeDtypeStruct((npad, 128), jnp.float32),
            jax.ShapeDtypeStruct((8, npad), jnp.float32),
            jax.ShapeDtypeStruct((8, npad), jnp.float32),
        ],
    )(x, w, a1, a2)

    if csplit:
        assert nc == 1
        half = cblk // 2
        out = pl.pallas_call(
            functools.partial(_stage2_split_kernel, half, out_f),
            grid=(nr,),
            in_specs=[
                pl.BlockSpec((rblk, half), lambda r: (r, 0)),
                pl.BlockSpec((rblk, half), lambda r: (r, 1)),
                pl.BlockSpec((npad, hw), lambda r: (0, 0)),
                pl.BlockSpec((rblk, 128), lambda r: (r, 0)),
                pl.BlockSpec((8, cblk), lambda r: (0, 0)),
                pl.BlockSpec((8, cblk), lambda r: (0, 0)),
            ],
            out_specs=pl.BlockSpec((rblk, out_f), lambda r: (r, 0)),
            out_shape=jax.ShapeDtypeStruct((n, out_f), jnp.float32),
            compiler_params=pltpu.CompilerParams(
                dimension_semantics=("parallel",)),
        )(adj, adj, h, u, v, va)
        return out

    out_w = out_f if nc == 1 else hw
    out = pl.pallas_call(
        functools.partial(_stage2_kernel, nc, cblk, out_f),
        grid=(nr, nc),
        in_specs=[
            pl.BlockSpec((rblk, cblk), lambda r, c: (r, c)),
            pl.BlockSpec((npad, hw), lambda r, c: (0, 0)),
            pl.BlockSpec((rblk, 128), lambda r, c: (r, 0)),
            pl.BlockSpec((8, cblk), lambda r, c: (0, c)),
            pl.BlockSpec((8, cblk), lambda r, c: (0, c)),
        ],
        out_specs=pl.BlockSpec((rblk, out_w), lambda r, c: (r, 0)),
        out_shape=jax.ShapeDtypeStruct(
            (n if nc == 1 else npad, out_w), jnp.float32),
        compiler_params=pltpu.CompilerParams(
            dimension_semantics=("parallel", "arbitrary")),
    )(adj, h, u, v, va)
    return out if nc == 1 else out[:n, :out_f]


def kernel(input, adj, W, a):
    return _gat_forward(input, adj, W, a)


# speedup vs baseline: 1.0390x; 1.0059x over previous
import functools
import math

import jax
import jax.numpy as jnp
from jax.experimental import pallas as pl
from jax.experimental.pallas import tpu as pltpu

ALPHA = 0.2


def _stage1_kernel(n, rblk, x_ref, w_ref, a1_ref, a2_ref,
                   h_ref, u_ref, v_ref, va_ref):
    r = pl.program_id(0)
    rows = r * rblk + jax.lax.broadcasted_iota(jnp.int32, (rblk, 1), 0)
    valid = rows < n
    h = jnp.dot(x_ref[...], w_ref[...], preferred_element_type=jnp.float32)
    h = jnp.where(valid, h, 0.0)
    h_ref[:, : h.shape[1]] = h
    if h_ref.shape[1] > h.shape[1]:
        h_ref[:, h.shape[1] :] = jnp.broadcast_to(
            jnp.where(valid, 1.0, 0.0), (rblk, h_ref.shape[1] - h.shape[1]))
    s1 = jax.lax.dot_general(h, a1_ref[...], (((1,), (1,)), ((), ())),
                             preferred_element_type=jnp.float32)
    u = jnp.where(valid, jnp.exp(-s1), 0.0)
    ua = jnp.where(valid, jnp.exp(-ALPHA * s1), 0.0)
    lane = jax.lax.broadcasted_iota(jnp.int32, (1, u_ref.shape[1]), 1)
    u_ref[...] = jnp.where(lane == 0, u, jnp.where(lane == 1, ua, 0.0))
    s2 = jax.lax.dot_general(a2_ref[...], h, (((1,), (1,)), ((), ())),
                             preferred_element_type=jnp.float32)
    cols = r * rblk + jax.lax.broadcasted_iota(jnp.int32, (1, rblk), 1)
    cvalid = cols < n
    v = jnp.where(cvalid, jnp.exp(-s2), 0.0)
    va = jnp.where(cvalid, jnp.exp(-ALPHA * s2), 0.0)
    v_ref[...] = jnp.broadcast_to(v, v_ref.shape)
    va_ref[...] = jnp.broadcast_to(va, va_ref.shape)


def _stage2_kernel(nc, cblk, out_f, adj_ref, h_ref, u_ref,
                   v_ref, va_ref, out_ref):
    c = pl.program_id(1)
    u = u_ref[:, 0:1]
    ua = u_ref[:, 1:2]
    v = v_ref[0:1, :]
    va = va_ref[0:1, :]
    e = jnp.where(adj_ref[...] > 0.0,
                  jnp.minimum(u * v, ua * va), 0.0)
    h_blk = h_ref[pl.ds(c * cblk, cblk), :]
    partial = jnp.dot(e, h_blk, preferred_element_type=jnp.float32)

    if nc == 1:
        hp = partial[:, :out_f] / partial[:, out_f : out_f + 1]
        out_ref[...] = jnp.where(hp > 0, hp, jnp.exp(hp) - 1.0)
        return

    @pl.when(c == 0)
    def _init():
        out_ref[...] = partial

    @pl.when(c > 0)
    def _acc():
        out_ref[...] = out_ref[...] + partial

    @pl.when(c == nc - 1)
    def _fin():
        acc = out_ref[...]
        hp = acc[:, :out_f] / acc[:, out_f : out_f + 1]
        out_ref[:, :out_f] = jnp.where(hp > 0, hp, jnp.exp(hp) - 1.0)


def _gat_forward(x, adj, w, a, rblk1=512, rblk=320, cblk=10240):
    n, in_f = x.shape
    out_f = w.shape[1]
    hw = out_f + 8
    step = math.lcm(rblk, cblk, rblk1)
    npad = ((n + step - 1) // step) * step
    nr = -(-n // rblk)
    nc = npad // cblk
    a1 = a[:, :out_f]
    a2 = a[:, out_f:]

    h, u, v, va = pl.pallas_call(
        functools.partial(_stage1_kernel, n, rblk1),
        grid=(npad // rblk1,),
        in_specs=[
            pl.BlockSpec((rblk1, in_f), lambda r: (r, 0)),
            pl.BlockSpec((in_f, out_f), lambda r: (0, 0)),
            pl.BlockSpec((1, out_f), lambda r: (0, 0)),
            pl.BlockSpec((1, out_f), lambda r: (0, 0)),
        ],
        out_specs=[
            pl.BlockSpec((rblk1, hw), lambda r: (r, 0)),
            pl.BlockSpec((rblk1, 128), lambda r: (r, 0)),
            pl.BlockSpec((8, rblk1), lambda r: (0, r)),
            pl.BlockSpec((8, rblk1), lambda r: (0, r)),
        ],
        out_shape=[
            jax.ShapeDtypeStruct((npad, hw), jnp.float32),
            jax.ShapeDtypeStruct((npad, 128), jnp.float32),
            jax.ShapeDtypeStruct((8, npad), jnp.float32),
            jax.ShapeDtypeStruct((8, npad), jnp.float32),
        ],
    )(x, w, a1, a2)

    out_w = out_f if nc == 1 else hw
    out = pl.pallas_call(
        functools.partial(_stage2_kernel, nc, cblk, out_f),
        grid=(nr, nc),
        in_specs=[
            pl.BlockSpec((rblk, cblk), lambda r, c: (r, c)),
            pl.BlockSpec((npad, hw), lambda r, c: (0, 0)),
            pl.BlockSpec((rblk, 128), lambda r, c: (r, 0)),
            pl.BlockSpec((8, cblk), lambda r, c: (0, c)),
            pl.BlockSpec((8, cblk), lambda r, c: (0, c)),
        ],
        out_specs=pl.BlockSpec((rblk, out_w), lambda r, c: (r, 0)),
        out_shape=jax.ShapeDtypeStruct(
            (n if nc == 1 else npad, out_w), jnp.float32),
        compiler_params=pltpu.CompilerParams(
            dimension_semantics=("parallel", "arbitrary")),
    )(adj, h, u, v, va)
    return out if nc == 1 else out[:n, :out_f]


def kernel(input, adj, W, a):
    return _gat_forward(input, adj, W, a)
